# Initial kernel scaffold; baseline (speedup 1.0000x reference)
import functools

import jax
import jax.numpy as jnp
from jax import lax
from jax.experimental import pallas as pl
from jax.experimental.pallas import tpu as pltpu
from jax.experimental.pallas import tpu_sc as plsc

_G = 8
_F = 128
_NQ = 4
_QC = _F // _NQ
_WIN = 128
_NTILES = 16
_NCORES = 2


def _ceil_div(a, b):
    return (a + b - 1) // b


@functools.partial(jax.jit, static_argnums=(3, 4, 5))
def _sc_segment_sum(h4, src_pad, dst_pad, n_pad, n_win_tile, rows_tile):
    mesh = plsc.VectorSubcoreMesh(core_axis_name="c", subcore_axis_name="s")
    zq = jnp.zeros((rows_tile, _QC), jnp.float32)

    @functools.partial(
        pl.kernel,
        out_type=jax.ShapeDtypeStruct((_NQ, n_pad, _QC), jnp.float32),
        mesh=mesh,
        scratch_types=[
            pltpu.VMEM((_WIN,), jnp.int32),
            pltpu.VMEM((_WIN,), jnp.int32),
            pltpu.VMEM((_WIN, _QC), jnp.float32),
            pltpu.VMEM((rows_tile, _QC), jnp.float32),
            pltpu.VMEM_SHARED((n_pad, _QC), jnp.float32),
            pltpu.SemaphoreType.DMA,
        ],
    )
    def segsum(h_hbm, src_hbm, dst_hbm, z_hbm, out_hbm,
               sidx_v, didx_v, rows_v, stage_v, acc_sh, sem):
        c = lax.axis_index("c")
        t = lax.axis_index("s")

        for r in range(2):
            q = c * 2 + r
            pltpu.sync_copy(z_hbm, stage_v)
            pltpu.sync_copy(stage_v, acc_sh.at[pl.ds(t * rows_tile, rows_tile), :])
            plsc.subcore_barrier()

            def body(w, carry):
                base = (t * n_win_tile + w) * _WIN
                pltpu.sync_copy(src_hbm.at[pl.ds(base, _WIN)], sidx_v)
                pltpu.sync_copy(dst_hbm.at[pl.ds(base, _WIN)], didx_v)
                pltpu.async_copy(h_hbm.at[q].at[sidx_v], rows_v, sem).wait()
                pltpu.sync_copy(rows_v, acc_sh.at[didx_v], add=True)
                return carry

            lax.fori_loop(0, n_win_tile, body, 0)
            plsc.subcore_barrier()
            sl = pl.ds(t * rows_tile, rows_tile)
            pltpu.sync_copy(acc_sh.at[sl, :], stage_v)
            pltpu.sync_copy(stage_v, out_hbm.at[q].at[sl, :])
            plsc.subcore_barrier()

    return segsum(h4, src_pad, dst_pad, zq)


def _to_quarters(h, n_pad):
    n = h.shape[0]
    h4 = h.reshape(n, _NQ, _QC).transpose(1, 0, 2)
    return jnp.pad(h4, ((0, 0), (0, n_pad - n), (0, 0)))


def _topk_mask(score, mask, batch, starts, ks_arr):
    n = score.shape[0]
    score = jnp.where(mask > 0, score, -jnp.inf)
    idx = jnp.arange(n)
    order = jnp.lexsort((idx, -score, batch))
    grp = jnp.take(batch, order)
    rank = idx - jnp.take(starts, grp)
    sel = rank < jnp.take(ks_arr, grp)
    new_mask = jnp.zeros_like(mask).at[order].set(sel.astype(mask.dtype))
    gate = jnp.tanh(jnp.where(mask > 0, score, 0.0))
    return gate, new_mask


def kernel(x, edge_index, batch, item_emb, cat_emb,
           W_rel1, b_rel1, W_root1, pw1,
           W_rel2, b_rel2, W_root2, pw2,
           W_rel3, b_rel3, W_root3, pw3,
           fc1_W, fc1_b, bn1_g, bn1_b, fc2_W, fc2_b, bn2_g, bn2_b):
    n = x.shape[0]
    e = edge_index.shape[1]

    rows_tile = _ceil_div(n, _NTILES * 8) * 8
    n_pad = rows_tile * _NTILES
    n_win_tile = _ceil_div(e, _NTILES * _WIN)
    e_pad = n_win_tile * _NTILES * _WIN

    src = edge_index[0].astype(jnp.int32)
    dst = edge_index[1].astype(jnp.int32)
    pad = e_pad - e
    src_pad = jnp.concatenate([src, jnp.zeros((pad,), jnp.int32)])
    dst_pad = jnp.concatenate(
        [dst, (n + (jnp.arange(pad, dtype=jnp.int32) % 8)) % n_pad])

    counts = jax.ops.segment_sum(jnp.ones_like(batch), batch, num_segments=_G)
    starts = jnp.cumsum(counts) - counts
    k1 = (9 * counts + 9) // 10
    k2 = (9 * k1 + 9) // 10
    k3 = (9 * k2 + 9) // 10

    item_id = x[:, 0, 0]
    cat_id = x[:, 0, 1]
    emb_item = jnp.take(item_emb, item_id, axis=0)
    emb_cat = jnp.take(cat_emb, cat_id, axis=0)
    h = jnp.concatenate([emb_item, emb_cat], axis=1)
    mask = jnp.ones((n,), dtype=h.dtype)

    def gpool(h, mask, cnts):
        neg = jnp.where(mask[:, None] > 0, h, -jnp.inf)
        mx = jax.ops.segment_max(neg, batch, num_segments=_G)
        mx = jnp.where(jnp.isfinite(mx), mx, 0.0)
        sm = jax.ops.segment_sum(h * mask[:, None], batch, num_segments=_G)
        cn = jnp.maximum(jnp.asarray(cnts, dtype=h.dtype), 1.0)[:, None]
        return jnp.concatenate([mx, sm / cn], axis=1)

    def bn(z, g, b):
        mu = jnp.mean(z, axis=0)
        var = jnp.mean((z - mu) ** 2, axis=0)
        return (z - mu) / jnp.sqrt(var + 1e-5) * g + b

    xs = []
    for (W_rel, b_rel, W_root, pw, ks) in (
            (W_rel1, b_rel1, W_root1, pw1, k1),
            (W_rel2, b_rel2, W_root2, pw2, k2),
            (W_rel3, b_rel3, W_root3, pw3, k3)):
        h4 = _to_quarters(h, n_pad)
        agg4 = _sc_segment_sum(h4, src_pad, dst_pad, n_pad, n_win_tile, rows_tile)
        agg = agg4[:, :n].transpose(1, 0, 2).reshape(n, _F)
        hc = jax.nn.relu(agg @ W_rel.T + b_rel + h @ W_root.T)
        score = (hc @ pw) / jnp.linalg.norm(pw)
        gate, mask = _topk_mask(score, mask, batch, starts, ks)
        h = hc * gate[:, None] * mask[:, None]
        xs.append(gpool(h, mask, ks))

    xg = xs[0] + xs[1] + xs[2]
    z = jax.nn.relu(bn(xg @ fc1_W.T + fc1_b, bn1_g, bn1_b))
    z = jax.nn.relu(bn(z @ fc2_W.T + fc2_b, bn2_g, bn2_b))
    z = jax.nn.relu(z)
    per_node = jnp.take(z, batch, axis=0)
    out = jax.nn.sigmoid(jnp.sum(emb_item * per_node, axis=1))
    return out


# speedup vs baseline: 3.7426x; 3.7426x over previous
import functools

import jax
import jax.numpy as jnp
from jax import lax
from jax.experimental import pallas as pl
from jax.experimental.pallas import tpu as pltpu
from jax.experimental.pallas import tpu_sc as plsc

_G = 8
_F = 128
_NQ = 8
_QC = _F // _NQ
_WIN = 128
_NTILES = 16
_NCORES = 2


def _ceil_div(a, b):
    return (a + b - 1) // b


@functools.partial(jax.jit, static_argnums=(3, 4, 5))
def _sc_segment_sum(h4, src_pad, dst_pad, n_pad, n_win_tile, rows_tile):
    mesh = plsc.VectorSubcoreMesh(core_axis_name="c", subcore_axis_name="s")
    zq = jnp.zeros((rows_tile, _QC), jnp.float32)

    @functools.partial(
        pl.kernel,
        out_type=jax.ShapeDtypeStruct((_NQ, n_pad, _QC), jnp.float32),
        mesh=mesh,
        compiler_params=pltpu.CompilerParams(use_tc_tiling_on_sc=False),
        scratch_types=[
            pltpu.VMEM((_WIN,), jnp.int32),
            pltpu.VMEM((_WIN,), jnp.int32),
            pltpu.VMEM((_WIN, _QC), jnp.float32),
            pltpu.VMEM((rows_tile, _QC), jnp.float32),
            pltpu.VMEM_SHARED((n_pad, _QC), jnp.float32),
            pltpu.SemaphoreType.DMA,
        ],
    )
    def segsum(h_hbm, src_hbm, dst_hbm, z_hbm, out_hbm,
               sidx_v, didx_v, rows_v, stage_v, acc_sh, sem):
        c = lax.axis_index("c")
        t = lax.axis_index("s")

        for r in range(_NQ // _NCORES):
            q = c * (_NQ // _NCORES) + r
            pltpu.sync_copy(z_hbm, stage_v)
            pltpu.sync_copy(stage_v, acc_sh.at[pl.ds(t * rows_tile, rows_tile), :])
            plsc.subcore_barrier()

            @pl.loop(0, n_win_tile)
            def _body(w):
                base = (t * n_win_tile + w) * _WIN
                pltpu.sync_copy(src_hbm.at[pl.ds(base, _WIN)], sidx_v)
                pltpu.sync_copy(dst_hbm.at[pl.ds(base, _WIN)], didx_v)
                pltpu.async_copy(h_hbm.at[q].at[sidx_v], rows_v, sem).wait()
                pltpu.sync_copy(rows_v, acc_sh.at[didx_v], add=True)
            plsc.subcore_barrier()
            sl = pl.ds(t * rows_tile, rows_tile)
            pltpu.sync_copy(acc_sh.at[sl, :], stage_v)
            pltpu.sync_copy(stage_v, out_hbm.at[q].at[sl, :])
            plsc.subcore_barrier()

    return segsum(h4, src_pad, dst_pad, zq)


def _to_quarters(h, n_pad):
    n = h.shape[0]
    h4 = h.reshape(n, _NQ, _QC).transpose(1, 0, 2)
    return jnp.pad(h4, ((0, 0), (0, n_pad - n), (0, 0)))


def _topk_mask(score, mask, batch, starts, ks_arr):
    n = score.shape[0]
    score = jnp.where(mask > 0, score, -jnp.inf)
    idx = jnp.arange(n)
    order = jnp.lexsort((idx, -score, batch))
    grp = jnp.take(batch, order)
    rank = idx - jnp.take(starts, grp)
    sel = rank < jnp.take(ks_arr, grp)
    new_mask = jnp.zeros_like(mask).at[order].set(sel.astype(mask.dtype))
    gate = jnp.tanh(jnp.where(mask > 0, score, 0.0))
    return gate, new_mask


def kernel(x, edge_index, batch, item_emb, cat_emb,
           W_rel1, b_rel1, W_root1, pw1,
           W_rel2, b_rel2, W_root2, pw2,
           W_rel3, b_rel3, W_root3, pw3,
           fc1_W, fc1_b, bn1_g, bn1_b, fc2_W, fc2_b, bn2_g, bn2_b):
    n = x.shape[0]
    e = edge_index.shape[1]

    rows_tile = _ceil_div(n, _NTILES * 8) * 8
    n_pad = rows_tile * _NTILES
    n_win_tile = _ceil_div(e, _NTILES * _WIN)
    e_pad = n_win_tile * _NTILES * _WIN

    src = edge_index[0].astype(jnp.int32)
    dst = edge_index[1].astype(jnp.int32)
    pad = e_pad - e
    src_pad = jnp.concatenate([src, jnp.zeros((pad,), jnp.int32)])
    dst_pad = jnp.concatenate(
        [dst, (n + (jnp.arange(pad, dtype=jnp.int32) % 8)) % n_pad])

    counts = jax.ops.segment_sum(jnp.ones_like(batch), batch, num_segments=_G)
    starts = jnp.cumsum(counts) - counts
    k1 = (9 * counts + 9) // 10
    k2 = (9 * k1 + 9) // 10
    k3 = (9 * k2 + 9) // 10

    item_id = x[:, 0, 0]
    cat_id = x[:, 0, 1]
    emb_item = jnp.take(item_emb, item_id, axis=0)
    emb_cat = jnp.take(cat_emb, cat_id, axis=0)
    h = jnp.concatenate([emb_item, emb_cat], axis=1)
    mask = jnp.ones((n,), dtype=h.dtype)

    def gpool(h, mask, cnts):
        neg = jnp.where(mask[:, None] > 0, h, -jnp.inf)
        mx = jax.ops.segment_max(neg, batch, num_segments=_G)
        mx = jnp.where(jnp.isfinite(mx), mx, 0.0)
        sm = jax.ops.segment_sum(h * mask[:, None], batch, num_segments=_G)
        cn = jnp.maximum(jnp.asarray(cnts, dtype=h.dtype), 1.0)[:, None]
        return jnp.concatenate([mx, sm / cn], axis=1)

    def bn(z, g, b):
        mu = jnp.mean(z, axis=0)
        var = jnp.mean((z - mu) ** 2, axis=0)
        return (z - mu) / jnp.sqrt(var + 1e-5) * g + b

    xs = []
    for (W_rel, b_rel, W_root, pw, ks) in (
            (W_rel1, b_rel1, W_root1, pw1, k1),
            (W_rel2, b_rel2, W_root2, pw2, k2),
            (W_rel3, b_rel3, W_root3, pw3, k3)):
        h4 = _to_quarters(h, n_pad)
        agg4 = _sc_segment_sum(h4, src_pad, dst_pad, n_pad, n_win_tile, rows_tile)
        agg = agg4[:, :n].transpose(1, 0, 2).reshape(n, _F)
        hc = jax.nn.relu(agg @ W_rel.T + b_rel + h @ W_root.T)
        score = (hc @ pw) / jnp.linalg.norm(pw)
        gate, mask = _topk_mask(score, mask, batch, starts, ks)
        h = hc * gate[:, None] * mask[:, None]
        xs.append(gpool(h, mask, ks))

    xg = xs[0] + xs[1] + xs[2]
    z = jax.nn.relu(bn(xg @ fc1_W.T + fc1_b, bn1_g, bn1_b))
    z = jax.nn.relu(bn(z @ fc2_W.T + fc2_b, bn2_g, bn2_b))
    z = jax.nn.relu(z)
    per_node = jnp.take(z, batch, axis=0)
    out = jax.nn.sigmoid(jnp.sum(emb_item * per_node, axis=1))
    return out


# speedup vs baseline: 7.0758x; 1.8906x over previous
import functools

import jax
import jax.numpy as jnp
from jax import lax
from jax.experimental import pallas as pl
from jax.experimental.pallas import tpu as pltpu
from jax.experimental.pallas import tpu_sc as plsc

_G = 8
_F = 128
_NQ = 8
_QC = _F // _NQ
_WIN = 128
_NTILES = 16
_NCORES = 2


def _ceil_div(a, b):
    return (a + b - 1) // b


_GK = 8


@functools.partial(jax.jit, static_argnums=(3, 4, 5))
def _sc_segment_sum(h4, src2, dst2, n_pad, n_win_tile, rows_tile):
    mesh = plsc.VectorSubcoreMesh(core_axis_name="c", subcore_axis_name="s")
    zq = jnp.zeros((rows_tile, _QC), jnp.float32)

    @functools.partial(
        pl.kernel,
        out_type=jax.ShapeDtypeStruct((_NQ, n_pad, _QC), jnp.float32),
        mesh=mesh,
        compiler_params=pltpu.CompilerParams(use_tc_tiling_on_sc=False),
        scratch_types=[
            pltpu.VMEM((_GK, _WIN), jnp.int32),
            pltpu.VMEM((_GK, _WIN), jnp.int32),
            pltpu.VMEM((_GK, _WIN, _QC), jnp.float32),
            pltpu.VMEM((rows_tile, _QC), jnp.float32),
            pltpu.VMEM_SHARED((n_pad, _QC), jnp.float32),
            pltpu.SemaphoreType.DMA,
            pltpu.SemaphoreType.DMA,
        ],
    )
    def segsum(h_hbm, src_hbm, dst_hbm, z_hbm, out_hbm,
               sidx_v, didx_v, rows_v, stage_v, acc_sh, gsem, ssem):
        c = lax.axis_index("c")
        t = lax.axis_index("s")

        for r in range(_NQ // _NCORES):
            q = c * (_NQ // _NCORES) + r
            pltpu.sync_copy(z_hbm, stage_v)
            pltpu.sync_copy(stage_v, acc_sh.at[pl.ds(t * rows_tile, rows_tile), :])
            plsc.subcore_barrier()

            @pl.loop(0, n_win_tile // _GK)
            def _body(g):
                w0 = t * n_win_tile + g * _GK
                pltpu.sync_copy(src_hbm.at[pl.ds(w0, _GK), :], sidx_v)
                pltpu.sync_copy(dst_hbm.at[pl.ds(w0, _GK), :], didx_v)
                hs = [pltpu.async_copy(h_hbm.at[q].at[sidx_v.at[b]],
                                       rows_v.at[b], gsem)
                      for b in range(_GK)]
                for hd in hs:
                    hd.wait()
                ss = [pltpu.async_copy(rows_v.at[b], acc_sh.at[didx_v.at[b]],
                                       ssem, add=True)
                      for b in range(_GK)]
                for hd in ss:
                    hd.wait()
            plsc.subcore_barrier()
            sl = pl.ds(t * rows_tile, rows_tile)
            pltpu.sync_copy(acc_sh.at[sl, :], stage_v)
            pltpu.sync_copy(stage_v, out_hbm.at[q].at[sl, :])
            plsc.subcore_barrier()

    return segsum(h4, src2, dst2, zq)


def _to_quarters(h, n_pad):
    n = h.shape[0]
    h4 = h.reshape(n, _NQ, _QC).transpose(1, 0, 2)
    return jnp.pad(h4, ((0, 0), (0, n_pad - n), (0, 0)))


def _topk_mask(score, mask, batch, starts, ks_arr):
    n = score.shape[0]
    score = jnp.where(mask > 0, score, -jnp.inf)
    idx = jnp.arange(n)
    order = jnp.lexsort((idx, -score, batch))
    grp = jnp.take(batch, order)
    rank = idx - jnp.take(starts, grp)
    sel = rank < jnp.take(ks_arr, grp)
    new_mask = jnp.zeros_like(mask).at[order].set(sel.astype(mask.dtype))
    gate = jnp.tanh(jnp.where(mask > 0, score, 0.0))
    return gate, new_mask


def kernel(x, edge_index, batch, item_emb, cat_emb,
           W_rel1, b_rel1, W_root1, pw1,
           W_rel2, b_rel2, W_root2, pw2,
           W_rel3, b_rel3, W_root3, pw3,
           fc1_W, fc1_b, bn1_g, bn1_b, fc2_W, fc2_b, bn2_g, bn2_b):
    n = x.shape[0]
    e = edge_index.shape[1]

    rows_tile = _ceil_div(n, _NTILES * 8) * 8
    n_pad = rows_tile * _NTILES
    n_win_tile = _ceil_div(e, _NTILES * _WIN * _GK) * _GK
    e_pad = n_win_tile * _NTILES * _WIN

    src = edge_index[0].astype(jnp.int32)
    dst = edge_index[1].astype(jnp.int32)
    pad = e_pad - e
    src_pad = jnp.concatenate([src, jnp.zeros((pad,), jnp.int32)])
    dst_pad = jnp.concatenate(
        [dst, (n + (jnp.arange(pad, dtype=jnp.int32) % 8)) % n_pad])
    src2 = src_pad.reshape(_NTILES * n_win_tile, _WIN)
    dst2 = dst_pad.reshape(_NTILES * n_win_tile, _WIN)

    counts = jax.ops.segment_sum(jnp.ones_like(batch), batch, num_segments=_G)
    starts = jnp.cumsum(counts) - counts
    k1 = (9 * counts + 9) // 10
    k2 = (9 * k1 + 9) // 10
    k3 = (9 * k2 + 9) // 10

    item_id = x[:, 0, 0]
    cat_id = x[:, 0, 1]
    emb_item = jnp.take(item_emb, item_id, axis=0)
    emb_cat = jnp.take(cat_emb, cat_id, axis=0)
    h = jnp.concatenate([emb_item, emb_cat], axis=1)
    mask = jnp.ones((n,), dtype=h.dtype)

    def gpool(h, mask, cnts):
        neg = jnp.where(mask[:, None] > 0, h, -jnp.inf)
        mx = jax.ops.segment_max(neg, batch, num_segments=_G)
        mx = jnp.where(jnp.isfinite(mx), mx, 0.0)
        sm = jax.ops.segment_sum(h * mask[:, None], batch, num_segments=_G)
        cn = jnp.maximum(jnp.asarray(cnts, dtype=h.dtype), 1.0)[:, None]
        return jnp.concatenate([mx, sm / cn], axis=1)

    def bn(z, g, b):
        mu = jnp.mean(z, axis=0)
        var = jnp.mean((z - mu) ** 2, axis=0)
        return (z - mu) / jnp.sqrt(var + 1e-5) * g + b

    xs = []
    for (W_rel, b_rel, W_root, pw, ks) in (
            (W_rel1, b_rel1, W_root1, pw1, k1),
            (W_rel2, b_rel2, W_root2, pw2, k2),
            (W_rel3, b_rel3, W_root3, pw3, k3)):
        h4 = _to_quarters(h, n_pad)
        agg4 = _sc_segment_sum(h4, src2, dst2, n_pad, n_win_tile, rows_tile)
        agg = agg4[:, :n].transpose(1, 0, 2).reshape(n, _F)
        hc = jax.nn.relu(agg @ W_rel.T + b_rel + h @ W_root.T)
        score = (hc @ pw) / jnp.linalg.norm(pw)
        gate, mask = _topk_mask(score, mask, batch, starts, ks)
        h = hc * gate[:, None] * mask[:, None]
        xs.append(gpool(h, mask, ks))

    xg = xs[0] + xs[1] + xs[2]
    z = jax.nn.relu(bn(xg @ fc1_W.T + fc1_b, bn1_g, bn1_b))
    z = jax.nn.relu(bn(z @ fc2_W.T + fc2_b, bn2_g, bn2_b))
    z = jax.nn.relu(z)
    per_node = jnp.take(z, batch, axis=0)
    out = jax.nn.sigmoid(jnp.sum(emb_item * per_node, axis=1))
    return out


# speedup vs baseline: 8.8837x; 1.2555x over previous
import functools

import jax
import jax.numpy as jnp
from jax import lax
from jax.experimental import pallas as pl
from jax.experimental.pallas import tpu as pltpu
from jax.experimental.pallas import tpu_sc as plsc

_G = 8
_F = 128
_NQ = 8
_QC = _F // _NQ
_WIN = 128
_NTILES = 16
_NCORES = 2


def _ceil_div(a, b):
    return (a + b - 1) // b


_GK = 8


@functools.partial(jax.jit, static_argnums=(3, 4, 5))
def _sc_segment_sum(h4, src2, dst2, n_pad, n_win_tile, rows_tile):
    mesh = plsc.VectorSubcoreMesh(core_axis_name="c", subcore_axis_name="s")
    zq = jnp.zeros((rows_tile, _QC), jnp.float32)

    @functools.partial(
        pl.kernel,
        out_type=jax.ShapeDtypeStruct((_NQ, n_pad, _QC), jnp.float32),
        mesh=mesh,
        compiler_params=pltpu.CompilerParams(use_tc_tiling_on_sc=False),
        scratch_types=[
            pltpu.VMEM((_GK, _WIN), jnp.int32),
            pltpu.VMEM((_GK, _WIN), jnp.int32),
            pltpu.VMEM((_GK, _WIN, _QC), jnp.float32),
            pltpu.VMEM((rows_tile, _QC), jnp.float32),
            pltpu.VMEM_SHARED((n_pad, _QC), jnp.float32),
            pltpu.SemaphoreType.DMA,
            pltpu.SemaphoreType.DMA,
        ],
    )
    def segsum(h_hbm, src_hbm, dst_hbm, z_hbm, out_hbm,
               sidx_v, didx_v, rows_v, stage_v, acc_sh, gsem, ssem):
        c = lax.axis_index("c")
        t = lax.axis_index("s")

        for r in range(_NQ // _NCORES):
            q = c * (_NQ // _NCORES) + r
            pltpu.sync_copy(z_hbm, stage_v)
            pltpu.sync_copy(stage_v, acc_sh.at[pl.ds(t * rows_tile, rows_tile), :])
            plsc.subcore_barrier()

            @pl.loop(0, n_win_tile // _GK)
            def _body(g):
                w0 = t * n_win_tile + g * _GK
                pltpu.sync_copy(src_hbm.at[pl.ds(w0, _GK), :], sidx_v)
                pltpu.sync_copy(dst_hbm.at[pl.ds(w0, _GK), :], didx_v)
                hs = [pltpu.async_copy(h_hbm.at[q].at[sidx_v.at[b]],
                                       rows_v.at[b], gsem)
                      for b in range(_GK)]
                for hd in hs:
                    hd.wait()
                ss = [pltpu.async_copy(rows_v.at[b], acc_sh.at[didx_v.at[b]],
                                       ssem, add=True)
                      for b in range(_GK)]
                for hd in ss:
                    hd.wait()
            plsc.subcore_barrier()
            sl = pl.ds(t * rows_tile, rows_tile)
            pltpu.sync_copy(acc_sh.at[sl, :], stage_v)
            pltpu.sync_copy(stage_v, out_hbm.at[q].at[sl, :])
            plsc.subcore_barrier()

    return segsum(h4, src2, dst2, zq)


def _to_quarters(h, n_pad):
    n = h.shape[0]
    h4 = h.reshape(n, _NQ, _QC).transpose(1, 0, 2)
    return jnp.pad(h4, ((0, 0), (0, n_pad - n), (0, 0)))


def _topk_mask(score, mask, batch, starts, ks_arr):
    n = score.shape[0]
    score = jnp.where(mask > 0, score, -jnp.inf)
    bits = jax.lax.bitcast_convert_type(score, jnp.uint32)
    key_up = bits ^ jnp.where(bits >> 31 != 0,
                              jnp.uint32(0xFFFFFFFF), jnp.uint32(0x80000000))
    key_desc = key_up ^ jnp.uint32(0xFFFFFFFF)
    hi = (batch.astype(jnp.uint32) << 29) | (key_desc >> 3)
    lo = ((key_desc & 7) << 17) | jnp.arange(n, dtype=jnp.uint32)
    shi, slo = jax.lax.sort((hi, lo), num_keys=2)
    pos = jnp.clip(starts + ks_arr - 1, 0, n - 1)
    thr_hi = jnp.take(shi, pos)
    thr_lo = jnp.take(slo, pos)
    bh = jnp.take(thr_hi, batch)
    bl = jnp.take(thr_lo, batch)
    sel = ((hi < bh) | ((hi == bh) & (lo <= bl))) & (jnp.take(ks_arr, batch) > 0)
    new_mask = sel.astype(mask.dtype)
    gate = jnp.tanh(jnp.where(mask > 0, score, 0.0))
    return gate, new_mask


def kernel(x, edge_index, batch, item_emb, cat_emb,
           W_rel1, b_rel1, W_root1, pw1,
           W_rel2, b_rel2, W_root2, pw2,
           W_rel3, b_rel3, W_root3, pw3,
           fc1_W, fc1_b, bn1_g, bn1_b, fc2_W, fc2_b, bn2_g, bn2_b):
    n = x.shape[0]
    e = edge_index.shape[1]

    rows_tile = _ceil_div(n, _NTILES * 8) * 8
    n_pad = rows_tile * _NTILES
    n_win_tile = _ceil_div(e, _NTILES * _WIN * _GK) * _GK
    e_pad = n_win_tile * _NTILES * _WIN

    src = edge_index[0].astype(jnp.int32)
    dst = edge_index[1].astype(jnp.int32)
    pad = e_pad - e
    src_pad = jnp.concatenate([src, jnp.zeros((pad,), jnp.int32)])
    dst_pad = jnp.concatenate(
        [dst, (n + (jnp.arange(pad, dtype=jnp.int32) % 8)) % n_pad])
    src2 = src_pad.reshape(_NTILES * n_win_tile, _WIN)
    dst2 = dst_pad.reshape(_NTILES * n_win_tile, _WIN)

    counts = jnp.sum(batch[:, None] == jnp.arange(_G)[None, :], axis=0)
    starts = jnp.cumsum(counts) - counts
    k1 = (9 * counts + 9) // 10
    k2 = (9 * k1 + 9) // 10
    k3 = (9 * k2 + 9) // 10

    item_id = x[:, 0, 0]
    cat_id = x[:, 0, 1]
    emb_item = jnp.take(item_emb, item_id, axis=0, mode="clip")
    emb_cat = jnp.take(cat_emb, cat_id, axis=0, mode="clip")
    h = jnp.concatenate([emb_item, emb_cat], axis=1)
    mask = jnp.ones((n,), dtype=h.dtype)

    onehot = (batch[:, None] == jnp.arange(_G)[None, :]).astype(jnp.float32)

    def gpool(h, mask, cnts):
        neg = jnp.where(mask[:, None] > 0, h, -jnp.inf)
        mx = jax.ops.segment_max(neg, batch, num_segments=_G,
                                 indices_are_sorted=True)
        mx = jnp.where(jnp.isfinite(mx), mx, 0.0)
        sm = jnp.einsum("ng,nf->gf", onehot, h * mask[:, None])
        cn = jnp.maximum(jnp.asarray(cnts, dtype=h.dtype), 1.0)[:, None]
        return jnp.concatenate([mx, sm / cn], axis=1)

    def bn(z, g, b):
        mu = jnp.mean(z, axis=0)
        var = jnp.mean((z - mu) ** 2, axis=0)
        return (z - mu) / jnp.sqrt(var + 1e-5) * g + b

    xs = []
    for (W_rel, b_rel, W_root, pw, ks) in (
            (W_rel1, b_rel1, W_root1, pw1, k1),
            (W_rel2, b_rel2, W_root2, pw2, k2),
            (W_rel3, b_rel3, W_root3, pw3, k3)):
        h4 = _to_quarters(h, n_pad)
        agg4 = _sc_segment_sum(h4, src2, dst2, n_pad, n_win_tile, rows_tile)
        agg = agg4[:, :n].transpose(1, 0, 2).reshape(n, _F)
        hc = jax.nn.relu(agg @ W_rel.T + b_rel + h @ W_root.T)
        score = (hc @ pw) / jnp.linalg.norm(pw)
        gate, mask = _topk_mask(score, mask, batch, starts, ks)
        h = hc * gate[:, None] * mask[:, None]
        xs.append(gpool(h, mask, ks))

    xg = xs[0] + xs[1] + xs[2]
    z = jax.nn.relu(bn(xg @ fc1_W.T + fc1_b, bn1_g, bn1_b))
    z = jax.nn.relu(bn(z @ fc2_W.T + fc2_b, bn2_g, bn2_b))
    z = jax.nn.relu(z)
    per_node = jnp.take(z, batch, axis=0, mode="clip")
    out = jax.nn.sigmoid(jnp.sum(emb_item * per_node, axis=1))
    return out


# speedup vs baseline: 9.7942x; 1.1025x over previous
import functools

import jax
import jax.numpy as jnp
from jax import lax
from jax.experimental import pallas as pl
from jax.experimental.pallas import tpu as pltpu
from jax.experimental.pallas import tpu_sc as plsc

_G = 8
_F = 128
_NQ = 8
_QC = _F // _NQ
_WIN = 128
_NTILES = 16
_NCORES = 2


def _ceil_div(a, b):
    return (a + b - 1) // b


_GK = 8


@functools.partial(jax.jit, static_argnums=(3, 4, 5))
def _sc_segment_sum(h4, src2, dst2, n_pad, n_win_tile, rows_tile):
    mesh = plsc.VectorSubcoreMesh(core_axis_name="c", subcore_axis_name="s")
    zq = jnp.zeros((rows_tile, _QC), jnp.float32)

    @functools.partial(
        pl.kernel,
        out_type=jax.ShapeDtypeStruct((_NQ, n_pad, _QC), jnp.float32),
        mesh=mesh,
        compiler_params=pltpu.CompilerParams(use_tc_tiling_on_sc=False),
        scratch_types=[
            pltpu.VMEM((_GK, _WIN), jnp.int32),
            pltpu.VMEM((_GK, _WIN), jnp.int32),
            pltpu.VMEM((_GK, _WIN, _QC), jnp.float32),
            pltpu.VMEM((rows_tile, _QC), jnp.float32),
            pltpu.VMEM_SHARED((n_pad, _QC), jnp.float32),
            pltpu.SemaphoreType.DMA,
            pltpu.SemaphoreType.DMA,
        ],
    )
    def segsum(h_hbm, src_hbm, dst_hbm, z_hbm, out_hbm,
               sidx_v, didx_v, rows_v, stage_v, acc_sh, gsem, ssem):
        c = lax.axis_index("c")
        t = lax.axis_index("s")

        for r in range(_NQ // _NCORES):
            q = c * (_NQ // _NCORES) + r
            pltpu.sync_copy(z_hbm, stage_v)
            pltpu.sync_copy(stage_v, acc_sh.at[pl.ds(t * rows_tile, rows_tile), :])
            plsc.subcore_barrier()

            @pl.loop(0, n_win_tile // _GK)
            def _body(g):
                w0 = t * n_win_tile + g * _GK
                pltpu.sync_copy(src_hbm.at[pl.ds(w0, _GK), :], sidx_v)
                pltpu.sync_copy(dst_hbm.at[pl.ds(w0, _GK), :], didx_v)
                hs = [pltpu.async_copy(h_hbm.at[q].at[sidx_v.at[b]],
                                       rows_v.at[b], gsem)
                      for b in range(_GK)]
                for hd in hs:
                    hd.wait()
                ss = [pltpu.async_copy(rows_v.at[b], acc_sh.at[didx_v.at[b]],
                                       ssem, add=True)
                      for b in range(_GK)]
                for hd in ss:
                    hd.wait()
            plsc.subcore_barrier()
            sl = pl.ds(t * rows_tile, rows_tile)
            pltpu.sync_copy(acc_sh.at[sl, :], stage_v)
            pltpu.sync_copy(stage_v, out_hbm.at[q].at[sl, :])
            plsc.subcore_barrier()

    return segsum(h4, src2, dst2, zq)


def _to_quarters(h, n_pad):
    n = h.shape[0]
    h4 = h.reshape(n, _NQ, _QC).transpose(1, 0, 2)
    return jnp.pad(h4, ((0, 0), (0, n_pad - n), (0, 0)))


def _topk_mask(score, mask, batch, starts, ks_arr):
    n = score.shape[0]
    score = jnp.where(mask > 0, score, -jnp.inf)
    bits = jax.lax.bitcast_convert_type(score, jnp.uint32)
    key_up = bits ^ jnp.where(bits >> 31 != 0,
                              jnp.uint32(0xFFFFFFFF), jnp.uint32(0x80000000))
    key_desc = key_up ^ jnp.uint32(0xFFFFFFFF)
    hi = (batch.astype(jnp.uint32) << 29) | (key_desc >> 3)
    lo = ((key_desc & 7) << 17) | jnp.arange(n, dtype=jnp.uint32)
    shi, slo = jax.lax.sort((hi, lo), num_keys=2)
    pos = jnp.clip(starts + ks_arr - 1, 0, n - 1)
    thr_hi = jnp.take(shi, pos)
    thr_lo = jnp.take(slo, pos)
    bh = jnp.take(thr_hi, batch)
    bl = jnp.take(thr_lo, batch)
    sel = ((hi < bh) | ((hi == bh) & (lo <= bl))) & (jnp.take(ks_arr, batch) > 0)
    new_mask = sel.astype(mask.dtype)
    gate = jnp.tanh(jnp.where(mask > 0, score, 0.0))
    return gate, new_mask


def kernel(x, edge_index, batch, item_emb, cat_emb,
           W_rel1, b_rel1, W_root1, pw1,
           W_rel2, b_rel2, W_root2, pw2,
           W_rel3, b_rel3, W_root3, pw3,
           fc1_W, fc1_b, bn1_g, bn1_b, fc2_W, fc2_b, bn2_g, bn2_b):
    n = x.shape[0]
    e = edge_index.shape[1]

    rows_tile = _ceil_div(n, _NTILES * 8) * 8
    n_pad = rows_tile * _NTILES
    n_win_tile = _ceil_div(e, _NTILES * _WIN * _GK) * _GK
    e_pad = n_win_tile * _NTILES * _WIN

    src = edge_index[0].astype(jnp.int32)
    dst = edge_index[1].astype(jnp.int32)
    pad = e_pad - e
    src_pad = jnp.concatenate([src, jnp.zeros((pad,), jnp.int32)])
    dst_pad = jnp.concatenate(
        [dst, (n + (jnp.arange(pad, dtype=jnp.int32) % 8)) % n_pad])
    src2 = src_pad.reshape(_NTILES * n_win_tile, _WIN)
    dst2 = dst_pad.reshape(_NTILES * n_win_tile, _WIN)

    counts = jnp.sum(batch[:, None] == jnp.arange(_G)[None, :], axis=0)
    starts = jnp.cumsum(counts) - counts
    k1 = (9 * counts + 9) // 10
    k2 = (9 * k1 + 9) // 10
    k3 = (9 * k2 + 9) // 10

    item_id = x[:, 0, 0]
    cat_id = x[:, 0, 1]
    emb_item = jnp.take(item_emb, item_id, axis=0, mode="clip")
    emb_cat = jnp.take(cat_emb, cat_id, axis=0, mode="clip")
    h = jnp.concatenate([emb_item, emb_cat], axis=1)
    mask = jnp.ones((n,), dtype=h.dtype)

    onehot = (batch[:, None] == jnp.arange(_G)[None, :]).astype(jnp.float32)

    def gpool(h, mask, cnts):
        neg = jnp.where(mask[:, None] > 0, h, -jnp.inf)
        mx = jnp.stack([jnp.max(jnp.where(batch[:, None] == g, neg, -jnp.inf),
                                axis=0) for g in range(_G)])
        mx = jnp.where(jnp.isfinite(mx), mx, 0.0)
        sm = jnp.einsum("ng,nf->gf", onehot, h * mask[:, None])
        cn = jnp.maximum(jnp.asarray(cnts, dtype=h.dtype), 1.0)[:, None]
        return jnp.concatenate([mx, sm / cn], axis=1)

    def bn(z, g, b):
        mu = jnp.mean(z, axis=0)
        var = jnp.mean((z - mu) ** 2, axis=0)
        return (z - mu) / jnp.sqrt(var + 1e-5) * g + b

    xs = []
    for (W_rel, b_rel, W_root, pw, ks) in (
            (W_rel1, b_rel1, W_root1, pw1, k1),
            (W_rel2, b_rel2, W_root2, pw2, k2),
            (W_rel3, b_rel3, W_root3, pw3, k3)):
        h4 = _to_quarters(h, n_pad)
        agg4 = _sc_segment_sum(h4, src2, dst2, n_pad, n_win_tile, rows_tile)
        agg = agg4[:, :n].transpose(1, 0, 2).reshape(n, _F)
        hc = jax.nn.relu(agg @ W_rel.T + b_rel + h @ W_root.T)
        score = (hc @ pw) / jnp.linalg.norm(pw)
        gate, mask = _topk_mask(score, mask, batch, starts, ks)
        h = hc * gate[:, None] * mask[:, None]
        xs.append(gpool(h, mask, ks))

    xg = xs[0] + xs[1] + xs[2]
    z = jax.nn.relu(bn(xg @ fc1_W.T + fc1_b, bn1_g, bn1_b))
    z = jax.nn.relu(bn(z @ fc2_W.T + fc2_b, bn2_g, bn2_b))
    z = jax.nn.relu(z)
    per_node = jnp.take(z, batch, axis=0, mode="clip")
    out = jax.nn.sigmoid(jnp.sum(emb_item * per_node, axis=1))
    return out
